# baseline (speedup 1.0000x reference)
import functools
import jax
import jax.numpy as jnp
from jax import lax
from jax.experimental import pallas as pl
from jax.experimental.pallas import tpu as pltpu
from jax.experimental.pallas import tpu_sc as plsc

_N = 10000
_E = 160000
_TIN = 12
_T1 = 10
_T2 = 8
_H = 128
_TN = 1000
_NB = _N // _TN
_F32 = jnp.float32


def _tconv1_body(x_ref, wa_ref, wb_ref, wc_ref, ba_ref, bb_ref, bc_ref,
                 out_ref):
    for t in range(_T1):
        pa = jnp.zeros((_TN, _H), _F32)
        pb = jnp.zeros((_TN, _H), _F32)
        pc = jnp.zeros((_TN, _H), _F32)
        for k in range(3):
            xv = x_ref[:, t + k:t + k + 1]
            pa = pa + xv * wa_ref[k:k + 1, :]
            pb = pb + xv * wb_ref[k:k + 1, :]
            pc = pc + xv * wc_ref[k:k + 1, :]
        pa = pa + ba_ref[:]
        pb = pb + bb_ref[:]
        pc = pc + bc_ref[:]
        out_ref[t] = jnp.maximum(pa * jax.nn.sigmoid(pb) + pc, 0.0)


def _tconv1(x2, wa, wb, wc, ba, bb, bc):
    return pl.pallas_call(
        _tconv1_body,
        grid=(_NB,),
        in_specs=[
            pl.BlockSpec((_TN, 16), lambda i: (i, 0)),
            pl.BlockSpec((8, _H), lambda i: (0, 0)),
            pl.BlockSpec((8, _H), lambda i: (0, 0)),
            pl.BlockSpec((8, _H), lambda i: (0, 0)),
            pl.BlockSpec((1, _H), lambda i: (0, 0)),
            pl.BlockSpec((1, _H), lambda i: (0, 0)),
            pl.BlockSpec((1, _H), lambda i: (0, 0)),
        ],
        out_specs=pl.BlockSpec((_T1, _TN, _H), lambda i: (0, i, 0)),
        out_shape=jax.ShapeDtypeStruct((_T1, _N, _H), _F32),
    )(x2, wa, wb, wc, ba, bb, bc)


def _cheb_body(tx0_ref, tx1_ref, s2_ref, wch_ref, bch_ref, out_ref):
    tx0 = tx0_ref[0]
    tx1 = tx1_ref[0]
    tx2 = -2.0 * (s2_ref[0, 0] + s2_ref[0, 1]) - tx0
    acc = jnp.dot(tx0, wch_ref[0], preferred_element_type=_F32)
    acc = acc + jnp.dot(tx1, wch_ref[1], preferred_element_type=_F32)
    acc = acc + jnp.dot(tx2, wch_ref[2], preferred_element_type=_F32)
    out_ref[0] = jnp.maximum(acc + bch_ref[:], 0.0)


def _cheb_combine(tx0, tx1, s2, wch, bch2):
    return pl.pallas_call(
        _cheb_body,
        grid=(_T1, _NB),
        in_specs=[
            pl.BlockSpec((1, _TN, _H), lambda t, i: (t, i, 0)),
            pl.BlockSpec((1, _TN, _H), lambda t, i: (t, i, 0)),
            pl.BlockSpec((1, 2, _TN, _H), lambda t, i: (t, 0, i, 0)),
            pl.BlockSpec((3, _H, _H), lambda t, i: (0, 0, 0)),
            pl.BlockSpec((1, _H), lambda t, i: (0, 0)),
        ],
        out_specs=pl.BlockSpec((1, _TN, _H), lambda t, i: (t, i, 0)),
        out_shape=jax.ShapeDtypeStruct((_T1, _N, _H), _F32),
    )(tx0, tx1, s2, wch, bch2)


def _tconv2_body(tg_ref, wa_ref, wb_ref, wc_ref, ba_ref, bb_ref, bc_ref,
                 out_ref):
    for t in range(_T2):
        pa = jnp.zeros((_TN, _H), _F32)
        pb = jnp.zeros((_TN, _H), _F32)
        pc = jnp.zeros((_TN, _H), _F32)
        for k in range(3):
            g = tg_ref[t + k]
            pa = pa + jnp.dot(g, wa_ref[k], preferred_element_type=_F32)
            pb = pb + jnp.dot(g, wb_ref[k], preferred_element_type=_F32)
            pc = pc + jnp.dot(g, wc_ref[k], preferred_element_type=_F32)
        pa = pa + ba_ref[:]
        pb = pb + bb_ref[:]
        pc = pc + bc_ref[:]
        out_ref[t] = jnp.maximum(pa * jax.nn.sigmoid(pb) + pc, 0.0)


def _tconv2(tg, wa, wb, wc, ba, bb, bc):
    return pl.pallas_call(
        _tconv2_body,
        grid=(_NB,),
        in_specs=[
            pl.BlockSpec((_T1, _TN, _H), lambda i: (0, i, 0)),
            pl.BlockSpec((3, _H, _H), lambda i: (0, 0, 0)),
            pl.BlockSpec((3, _H, _H), lambda i: (0, 0, 0)),
            pl.BlockSpec((3, _H, _H), lambda i: (0, 0, 0)),
            pl.BlockSpec((1, _H), lambda i: (0, 0)),
            pl.BlockSpec((1, _H), lambda i: (0, 0)),
            pl.BlockSpec((1, _H), lambda i: (0, 0)),
        ],
        out_specs=pl.BlockSpec((_T2, _TN, _H), lambda i: (0, i, 0)),
        out_shape=jax.ShapeDtypeStruct((_T2, _N, _H), _F32),
    )(tg, wa, wb, wc, ba, bb, bc)


def _head_body(t2_ref, gamma_ref, beta_ref, wlin_ref, blin_ref, out_ref):
    v = t2_ref[:]
    m = jnp.mean(v, axis=(0, 2), keepdims=True)
    var = jnp.mean((v - m) ** 2, axis=(0, 2), keepdims=True)
    inv = jax.lax.rsqrt(var + 1e-5)
    g = gamma_ref[:].reshape(1, _TN, 1)
    b = beta_ref[:].reshape(1, _TN, 1)
    tn = (v - m) * inv * g + b
    h = jnp.mean(jnp.maximum(tn, 0.0), axis=0)
    out_ref[:] = jnp.dot(h, wlin_ref[:], preferred_element_type=_F32) \
        + blin_ref[:]


def _head(t2, gamma2, beta2, wlin, blin2):
    return pl.pallas_call(
        _head_body,
        grid=(_NB,),
        in_specs=[
            pl.BlockSpec((_T2, _TN, _H), lambda i: (0, i, 0)),
            pl.BlockSpec((_TN, 1), lambda i: (i, 0)),
            pl.BlockSpec((_TN, 1), lambda i: (i, 0)),
            pl.BlockSpec((_H, 12), lambda i: (0, 0)),
            pl.BlockSpec((1, 12), lambda i: (0, 0)),
        ],
        out_specs=pl.BlockSpec((_TN, 12), lambda i: (i, 0)),
        out_shape=jax.ShapeDtypeStruct((_N, 12), _F32),
    )(t2, gamma2, beta2, wlin, blin2)


_NW = 32
_EPAD = _NW * 5120
_EPW = _EPAD // _NW
_NCH = _EPW // 128
_NPSP = 640
_NPAD = _NPSP * 16
_MESH = plsc.VectorSubcoreMesh(core_axis_name="c", subcore_axis_name="s")


def _zero16():
    return jnp.zeros((16,), _F32)


def _sc_deg(srcf, ewf):
    @functools.partial(
        pl.kernel, mesh=_MESH,
        compiler_params=pltpu.CompilerParams(needs_layout_passes=False),
        out_type=jax.ShapeDtypeStruct((_NW, _N), _F32),
        scratch_types=[
            pltpu.VMEM((_EPW,), jnp.int32),
            pltpu.VMEM((_EPW,), _F32),
            pltpu.VMEM((_N,), _F32),
        ],
    )
    def k(src_hbm, ew_hbm, out_hbm, src_v, ew_v, acc_v):
        wid = lax.axis_index("s") * 2 + lax.axis_index("c")
        pltpu.sync_copy(src_hbm.at[wid], src_v)
        pltpu.sync_copy(ew_hbm.at[wid], ew_v)

        def zb(i, _):
            acc_v[pl.ds(i * 16, 16)] = _zero16()
            return 0
        lax.fori_loop(0, _N // 16, zb, 0)

        def eb(i, _):
            s16 = src_v[pl.ds(i * 16, 16)]
            w16 = ew_v[pl.ds(i * 16, 16)]
            plsc.addupdate_scatter(acc_v, [s16], w16)
            return 0
        lax.fori_loop(0, _EPW // 16, eb, 0)
        pltpu.sync_copy(acc_v, out_hbm.at[wid])

    return k(srcf, ewf)


def _sc_wn(srcf, dstf, ewf, dis):
    @functools.partial(
        pl.kernel, mesh=_MESH,
        compiler_params=pltpu.CompilerParams(needs_layout_passes=False),
        out_type=jax.ShapeDtypeStruct((_NW, _EPW), _F32),
        scratch_types=[
            pltpu.VMEM((_EPW,), jnp.int32),
            pltpu.VMEM((_EPW,), jnp.int32),
            pltpu.VMEM((_EPW,), _F32),
            pltpu.VMEM((_N,), _F32),
            pltpu.VMEM((_EPW,), _F32),
        ],
    )
    def k(src_hbm, dst_hbm, ew_hbm, dis_hbm, out_hbm,
          src_v, dst_v, ew_v, dis_v, wn_v):
        wid = lax.axis_index("s") * 2 + lax.axis_index("c")
        pltpu.sync_copy(dis_hbm, dis_v)
        pltpu.sync_copy(src_hbm.at[wid], src_v)
        pltpu.sync_copy(dst_hbm.at[wid], dst_v)
        pltpu.sync_copy(ew_hbm.at[wid], ew_v)

        def eb(i, _):
            s16 = src_v[pl.ds(i * 16, 16)]
            d16 = dst_v[pl.ds(i * 16, 16)]
            w16 = ew_v[pl.ds(i * 16, 16)]
            a = plsc.load_gather(dis_v, [s16])
            b = plsc.load_gather(dis_v, [d16])
            wn_v[pl.ds(i * 16, 16)] = a * w16 * b
            return 0
        lax.fori_loop(0, _EPW // 16, eb, 0)
        pltpu.sync_copy(wn_v, out_hbm.at[wid])

    return k(srcf, dstf, ewf, dis)


def _sc_spmm(u, src2, dst2, wnf):
    @functools.partial(
        pl.kernel, mesh=_MESH,
        compiler_params=pltpu.CompilerParams(needs_layout_passes=False),
        out_type=jax.ShapeDtypeStruct((2, _N, _H), _F32),
        scratch_types=[
            pltpu.VMEM((_NCH, 128), jnp.int32),
            pltpu.VMEM((_NCH, 128), jnp.int32),
            pltpu.VMEM((_EPW,), _F32),
            pltpu.VMEM((128, _H), _F32),
            pltpu.VMEM((8, _H), _F32),
            pltpu.VMEM_SHARED((_NPAD, _H), _F32),
            pltpu.SemaphoreType.DMA,
        ],
    )
    def k(u_hbm, src_hbm, dst_hbm, wn_hbm, out_hbm,
          src_v, dst_v, wn_v, rows_v, zb_v, acc_sp, sem):
        cid = lax.axis_index("c")
        sid = lax.axis_index("s")
        wid = sid * 2 + cid
        pltpu.sync_copy(src_hbm.at[wid], src_v)
        pltpu.sync_copy(dst_hbm.at[wid], dst_v)
        pltpu.sync_copy(wn_hbm.at[wid], wn_v)

        for i in range(8):
            for g in range(8):
                zb_v[i, pl.ds(g * 16, 16)] = _zero16()

        def zb(i, _):
            pltpu.sync_copy(zb_v, acc_sp.at[pl.ds(sid * _NPSP + i * 8, 8)])
            return 0
        lax.fori_loop(0, _NPSP // 8, zb, 0)
        plsc.subcore_barrier()

        ridx0 = lax.iota(jnp.int32, 16)

        def chunk(j, _):
            pltpu.async_copy(u_hbm.at[src_v.at[j]], rows_v, sem).wait()

            def grp(g, _):
                w16 = wn_v[pl.ds(j * 128 + g * 16, 16)]
                ridx = ridx0 + g * 16
                for col in range(_H):
                    cidx = jnp.full((16,), col, jnp.int32)
                    v = plsc.load_gather(rows_v, [ridx, cidx])
                    plsc.store_scatter(rows_v, [ridx, cidx], v * w16)
                return 0
            lax.fori_loop(0, 8, grp, 0)
            pltpu.sync_copy(rows_v, acc_sp.at[dst_v.at[j]], add=True)
            return 0
        lax.fori_loop(0, _NCH, chunk, 0)
        plsc.subcore_barrier()

        @pl.when(sid < 15)
        def _():
            base = pl.multiple_of(sid * _NPSP, 8)
            pltpu.sync_copy(acc_sp.at[pl.ds(base, _NPSP)],
                            out_hbm.at[cid, pl.ds(base, _NPSP)])

        @pl.when(sid == 15)
        def _():
            pltpu.sync_copy(acc_sp.at[pl.ds(15 * _NPSP, _N - 15 * _NPSP)],
                            out_hbm.at[cid, pl.ds(15 * _NPSP,
                                                  _N - 15 * _NPSP)])

    return k(u, src2, dst2, wnf)


def _dis_body(degp_ref, out_ref):
    d = jnp.sum(degp_ref[:], axis=0, keepdims=True)
    out_ref[:] = jnp.where(
        d > 0, jax.lax.rsqrt(jnp.where(d > 0, d, 1.0)), 0.0)


def _dis(degp):
    return pl.pallas_call(
        _dis_body,
        out_shape=jax.ShapeDtypeStruct((1, _N), _F32),
    )(degp)


def _negsum_body(p_ref, out_ref):
    out_ref[0] = -(p_ref[0, 0] + p_ref[0, 1])


def _negsum(parts):
    return pl.pallas_call(
        _negsum_body,
        grid=(_T1, _NB),
        in_specs=[pl.BlockSpec((1, 2, _TN, _H), lambda t, i: (t, 0, i, 0))],
        out_specs=pl.BlockSpec((1, _TN, _H), lambda t, i: (t, i, 0)),
        out_shape=jax.ShapeDtypeStruct((_T1, _N, _H), _F32),
    )(parts)


def kernel(x, edge_index, edge_weight, W1a, b1a, W1b, b1b, W1c, b1c, Wch, bch,
           W2a, b2a, W2b, b2b, W2c, b2c, gamma, beta, Wlin, blin):
    src = edge_index[0].astype(jnp.int32)
    dst = edge_index[1].astype(jnp.int32)
    ew = edge_weight.astype(_F32)

    npad = _EPAD - _E
    srcp = jnp.concatenate([src, jnp.zeros((npad,), jnp.int32)])
    dstp = jnp.concatenate([dst, jnp.zeros((npad,), jnp.int32)])
    ewp = jnp.concatenate([ew, jnp.zeros((npad,), _F32)])
    srcf = srcp.reshape(_NW, _EPW)
    dstf = dstp.reshape(_NW, _EPW)
    ewf = ewp.reshape(_NW, _EPW)
    src2 = srcp.reshape(_NW, _NCH, 128)
    dst2 = dstp.reshape(_NW, _NCH, 128)

    degp = _sc_deg(srcf, ewf)
    dis = _dis(degp).reshape(_N)
    wnf = _sc_wn(srcf, dstf, ewf, dis)

    def S(u):
        return jnp.stack([_sc_spmm(u[t], src2, dst2, wnf)
                          for t in range(_T1)])

    x2 = jnp.pad(x[0, :, :, 0].T, ((0, 0), (0, 16 - _TIN)))
    pad8 = lambda w: jnp.pad(w[:, 0, 0, :].T, ((0, 5), (0, 0)))
    h1 = _tconv1(x2, pad8(W1a), pad8(W1b), pad8(W1c),
                 b1a.reshape(1, _H), b1b.reshape(1, _H), b1c.reshape(1, _H))

    tx1 = _negsum(S(h1))
    s2 = S(tx1)
    tg = _cheb_combine(h1, tx1, s2, Wch, bch.reshape(1, _H))

    taps = lambda w: jnp.transpose(w[:, :, 0, :], (2, 1, 0))
    t2 = _tconv2(tg, taps(W2a), taps(W2b), taps(W2c),
                 b2a.reshape(1, _H), b2b.reshape(1, _H), b2c.reshape(1, _H))

    return _head(t2, gamma.reshape(_N, 1), beta.reshape(_N, 1),
                 Wlin, blin.reshape(1, 12))


# speedup vs baseline: 3.1418x; 3.1418x over previous
import functools
import jax
import jax.numpy as jnp
from jax import lax
from jax.experimental import pallas as pl
from jax.experimental.pallas import tpu as pltpu
from jax.experimental.pallas import tpu_sc as plsc

_N = 10000
_E = 160000
_TIN = 12
_T1 = 10
_T2 = 8
_H = 128
_TN = 1000
_NB = _N // _TN
_F32 = jnp.float32


def _tconv1_body(x_ref, wa_ref, wb_ref, wc_ref, ba_ref, bb_ref, bc_ref,
                 out_ref):
    for t in range(_T1):
        pa = jnp.zeros((_TN, _H), _F32)
        pb = jnp.zeros((_TN, _H), _F32)
        pc = jnp.zeros((_TN, _H), _F32)
        for k in range(3):
            xv = x_ref[:, t + k:t + k + 1]
            pa = pa + xv * wa_ref[k:k + 1, :]
            pb = pb + xv * wb_ref[k:k + 1, :]
            pc = pc + xv * wc_ref[k:k + 1, :]
        pa = pa + ba_ref[:]
        pb = pb + bb_ref[:]
        pc = pc + bc_ref[:]
        out_ref[t] = jnp.maximum(pa * jax.nn.sigmoid(pb) + pc, 0.0)


def _tconv1(x2, wa, wb, wc, ba, bb, bc):
    return pl.pallas_call(
        _tconv1_body,
        grid=(_NB,),
        in_specs=[
            pl.BlockSpec((_TN, 16), lambda i: (i, 0)),
            pl.BlockSpec((8, _H), lambda i: (0, 0)),
            pl.BlockSpec((8, _H), lambda i: (0, 0)),
            pl.BlockSpec((8, _H), lambda i: (0, 0)),
            pl.BlockSpec((1, _H), lambda i: (0, 0)),
            pl.BlockSpec((1, _H), lambda i: (0, 0)),
            pl.BlockSpec((1, _H), lambda i: (0, 0)),
        ],
        out_specs=pl.BlockSpec((_T1, _TN, _H), lambda i: (0, i, 0)),
        out_shape=jax.ShapeDtypeStruct((_T1, _N, _H), _F32),
    )(x2, wa, wb, wc, ba, bb, bc)


def _cheb_body(tx0_ref, tx1_ref, s2_ref, wch_ref, bch_ref, out_ref):
    tx0 = tx0_ref[0]
    tx1 = tx1_ref[0]
    tx2 = -2.0 * (s2_ref[0, 0] + s2_ref[0, 1]) - tx0
    acc = jnp.dot(tx0, wch_ref[0], preferred_element_type=_F32)
    acc = acc + jnp.dot(tx1, wch_ref[1], preferred_element_type=_F32)
    acc = acc + jnp.dot(tx2, wch_ref[2], preferred_element_type=_F32)
    out_ref[0] = jnp.maximum(acc + bch_ref[:], 0.0)


def _cheb_combine(tx0, tx1, s2, wch, bch2):
    return pl.pallas_call(
        _cheb_body,
        grid=(_T1, _NB),
        in_specs=[
            pl.BlockSpec((1, _TN, _H), lambda t, i: (t, i, 0)),
            pl.BlockSpec((1, _TN, _H), lambda t, i: (t, i, 0)),
            pl.BlockSpec((1, 2, _TN, _H), lambda t, i: (t, 0, i, 0)),
            pl.BlockSpec((3, _H, _H), lambda t, i: (0, 0, 0)),
            pl.BlockSpec((1, _H), lambda t, i: (0, 0)),
        ],
        out_specs=pl.BlockSpec((1, _TN, _H), lambda t, i: (t, i, 0)),
        out_shape=jax.ShapeDtypeStruct((_T1, _N, _H), _F32),
    )(tx0, tx1, s2, wch, bch2)


def _tconv2_body(tg_ref, wa_ref, wb_ref, wc_ref, ba_ref, bb_ref, bc_ref,
                 out_ref):
    for t in range(_T2):
        pa = jnp.zeros((_TN, _H), _F32)
        pb = jnp.zeros((_TN, _H), _F32)
        pc = jnp.zeros((_TN, _H), _F32)
        for k in range(3):
            g = tg_ref[t + k]
            pa = pa + jnp.dot(g, wa_ref[k], preferred_element_type=_F32)
            pb = pb + jnp.dot(g, wb_ref[k], preferred_element_type=_F32)
            pc = pc + jnp.dot(g, wc_ref[k], preferred_element_type=_F32)
        pa = pa + ba_ref[:]
        pb = pb + bb_ref[:]
        pc = pc + bc_ref[:]
        out_ref[t] = jnp.maximum(pa * jax.nn.sigmoid(pb) + pc, 0.0)


def _tconv2(tg, wa, wb, wc, ba, bb, bc):
    return pl.pallas_call(
        _tconv2_body,
        grid=(_NB,),
        in_specs=[
            pl.BlockSpec((_T1, _TN, _H), lambda i: (0, i, 0)),
            pl.BlockSpec((3, _H, _H), lambda i: (0, 0, 0)),
            pl.BlockSpec((3, _H, _H), lambda i: (0, 0, 0)),
            pl.BlockSpec((3, _H, _H), lambda i: (0, 0, 0)),
            pl.BlockSpec((1, _H), lambda i: (0, 0)),
            pl.BlockSpec((1, _H), lambda i: (0, 0)),
            pl.BlockSpec((1, _H), lambda i: (0, 0)),
        ],
        out_specs=pl.BlockSpec((_T2, _TN, _H), lambda i: (0, i, 0)),
        out_shape=jax.ShapeDtypeStruct((_T2, _N, _H), _F32),
    )(tg, wa, wb, wc, ba, bb, bc)


def _head_body(t2_ref, gamma_ref, beta_ref, wlin_ref, blin_ref, out_ref):
    v = t2_ref[:]
    m = jnp.mean(v, axis=(0, 2), keepdims=True)
    var = jnp.mean((v - m) ** 2, axis=(0, 2), keepdims=True)
    inv = jax.lax.rsqrt(var + 1e-5)
    g = gamma_ref[:].reshape(1, _TN, 1)
    b = beta_ref[:].reshape(1, _TN, 1)
    tn = (v - m) * inv * g + b
    h = jnp.mean(jnp.maximum(tn, 0.0), axis=0)
    out_ref[:] = jnp.dot(h, wlin_ref[:], preferred_element_type=_F32) \
        + blin_ref[:]


def _head(t2, gamma2, beta2, wlin, blin2):
    return pl.pallas_call(
        _head_body,
        grid=(_NB,),
        in_specs=[
            pl.BlockSpec((_T2, _TN, _H), lambda i: (0, i, 0)),
            pl.BlockSpec((_TN, 1), lambda i: (i, 0)),
            pl.BlockSpec((_TN, 1), lambda i: (i, 0)),
            pl.BlockSpec((_H, 12), lambda i: (0, 0)),
            pl.BlockSpec((1, 12), lambda i: (0, 0)),
        ],
        out_specs=pl.BlockSpec((_TN, 12), lambda i: (i, 0)),
        out_shape=jax.ShapeDtypeStruct((_N, 12), _F32),
    )(t2, gamma2, beta2, wlin, blin2)


_NW = 32
_EPAD = _NW * 5120
_EPW = _EPAD // _NW
_NCH = _EPW // 128
_NPSP = 640
_NPAD = _NPSP * 16
_MESH = plsc.VectorSubcoreMesh(core_axis_name="c", subcore_axis_name="s")


def _zero16():
    return jnp.zeros((16,), _F32)


def _sc_deg(srcf, ewf):
    @functools.partial(
        pl.kernel, mesh=_MESH,
        compiler_params=pltpu.CompilerParams(needs_layout_passes=False),
        out_type=jax.ShapeDtypeStruct((_NW, _N), _F32),
        scratch_types=[
            pltpu.VMEM((_EPW,), jnp.int32),
            pltpu.VMEM((_EPW,), _F32),
            pltpu.VMEM((_N,), _F32),
        ],
    )
    def k(src_hbm, ew_hbm, out_hbm, src_v, ew_v, acc_v):
        wid = lax.axis_index("s") * 2 + lax.axis_index("c")
        pltpu.sync_copy(src_hbm.at[wid], src_v)
        pltpu.sync_copy(ew_hbm.at[wid], ew_v)

        def zb(i, _):
            acc_v[pl.ds(i * 16, 16)] = _zero16()
            return 0
        lax.fori_loop(0, _N // 16, zb, 0)

        def eb(i, _):
            s16 = src_v[pl.ds(i * 16, 16)]
            w16 = ew_v[pl.ds(i * 16, 16)]
            plsc.addupdate_scatter(acc_v, [s16], w16)
            return 0
        lax.fori_loop(0, _EPW // 16, eb, 0)
        pltpu.sync_copy(acc_v, out_hbm.at[wid])

    return k(srcf, ewf)


def _sc_wn(srcf, dstf, ewf, dis):
    @functools.partial(
        pl.kernel, mesh=_MESH,
        compiler_params=pltpu.CompilerParams(needs_layout_passes=False),
        out_type=jax.ShapeDtypeStruct((_NW, _EPW), _F32),
        scratch_types=[
            pltpu.VMEM((_EPW,), jnp.int32),
            pltpu.VMEM((_EPW,), jnp.int32),
            pltpu.VMEM((_EPW,), _F32),
            pltpu.VMEM((_N,), _F32),
            pltpu.VMEM((_EPW,), _F32),
        ],
    )
    def k(src_hbm, dst_hbm, ew_hbm, dis_hbm, out_hbm,
          src_v, dst_v, ew_v, dis_v, wn_v):
        wid = lax.axis_index("s") * 2 + lax.axis_index("c")
        pltpu.sync_copy(dis_hbm, dis_v)
        pltpu.sync_copy(src_hbm.at[wid], src_v)
        pltpu.sync_copy(dst_hbm.at[wid], dst_v)
        pltpu.sync_copy(ew_hbm.at[wid], ew_v)

        def eb(i, _):
            s16 = src_v[pl.ds(i * 16, 16)]
            d16 = dst_v[pl.ds(i * 16, 16)]
            w16 = ew_v[pl.ds(i * 16, 16)]
            a = plsc.load_gather(dis_v, [s16])
            b = plsc.load_gather(dis_v, [d16])
            wn_v[pl.ds(i * 16, 16)] = a * w16 * b
            return 0
        lax.fori_loop(0, _EPW // 16, eb, 0)
        pltpu.sync_copy(wn_v, out_hbm.at[wid])

    return k(srcf, dstf, ewf, dis)


def _sc_spmm(u, src2, dst2, wnf):
    @functools.partial(
        pl.kernel, mesh=_MESH,
        compiler_params=pltpu.CompilerParams(needs_layout_passes=False),
        out_type=jax.ShapeDtypeStruct((2, _N, _H), _F32),
        scratch_types=[
            pltpu.VMEM((_NCH, 128), jnp.int32),
            pltpu.VMEM((_NCH, 128), jnp.int32),
            pltpu.VMEM((_EPW,), _F32),
            pltpu.VMEM((128, _H), _F32),
            pltpu.VMEM((8, _H), _F32),
            pltpu.VMEM_SHARED((_NPAD, _H), _F32),
            pltpu.SemaphoreType.DMA,
        ],
    )
    def k(u_hbm, src_hbm, dst_hbm, wn_hbm, out_hbm,
          src_v, dst_v, wn_v, rows_v, zb_v, acc_sp, sem):
        cid = lax.axis_index("c")
        sid = lax.axis_index("s")
        wid = sid * 2 + cid
        pltpu.sync_copy(src_hbm.at[wid], src_v)
        pltpu.sync_copy(dst_hbm.at[wid], dst_v)
        pltpu.sync_copy(wn_hbm.at[wid], wn_v)

        for i in range(8):
            for g in range(8):
                zb_v[i, pl.ds(g * 16, 16)] = _zero16()

        def zb(i, _):
            pltpu.sync_copy(zb_v, acc_sp.at[pl.ds(sid * _NPSP + i * 8, 8)])
            return 0
        lax.fori_loop(0, _NPSP // 8, zb, 0)
        plsc.subcore_barrier()

        def chunk(j, _):
            pltpu.async_copy(u_hbm.at[src_v.at[j]], rows_v, sem).wait()

            def grp(g, _):
                for r in range(16):
                    e = j * 128 + g * 16 + r
                    w16 = plsc.load_gather(
                        wn_v, [jnp.full((16,), e, jnp.int32)])
                    row = g * 16 + r
                    for c8 in range(8):
                        sl = pl.ds(c8 * 16, 16)
                        rows_v[row, sl] = rows_v[row, sl] * w16
                return 0
            lax.fori_loop(0, 8, grp, 0)
            pltpu.sync_copy(rows_v, acc_sp.at[dst_v.at[j]], add=True)
            return 0
        lax.fori_loop(0, _NCH, chunk, 0)
        plsc.subcore_barrier()

        @pl.when(sid < 15)
        def _():
            base = pl.multiple_of(sid * _NPSP, 8)
            pltpu.sync_copy(acc_sp.at[pl.ds(base, _NPSP)],
                            out_hbm.at[cid, pl.ds(base, _NPSP)])

        @pl.when(sid == 15)
        def _():
            pltpu.sync_copy(acc_sp.at[pl.ds(15 * _NPSP, _N - 15 * _NPSP)],
                            out_hbm.at[cid, pl.ds(15 * _NPSP,
                                                  _N - 15 * _NPSP)])

    return k(u, src2, dst2, wnf)


def _dis_body(degp_ref, out_ref):
    d = jnp.sum(degp_ref[:], axis=0, keepdims=True)
    out_ref[:] = jnp.where(
        d > 0, jax.lax.rsqrt(jnp.where(d > 0, d, 1.0)), 0.0)


def _dis(degp):
    return pl.pallas_call(
        _dis_body,
        out_shape=jax.ShapeDtypeStruct((1, _N), _F32),
    )(degp)


def _negsum_body(p_ref, out_ref):
    out_ref[0] = -(p_ref[0, 0] + p_ref[0, 1])


def _negsum(parts):
    return pl.pallas_call(
        _negsum_body,
        grid=(_T1, _NB),
        in_specs=[pl.BlockSpec((1, 2, _TN, _H), lambda t, i: (t, 0, i, 0))],
        out_specs=pl.BlockSpec((1, _TN, _H), lambda t, i: (t, i, 0)),
        out_shape=jax.ShapeDtypeStruct((_T1, _N, _H), _F32),
    )(parts)


def kernel(x, edge_index, edge_weight, W1a, b1a, W1b, b1b, W1c, b1c, Wch, bch,
           W2a, b2a, W2b, b2b, W2c, b2c, gamma, beta, Wlin, blin):
    src = edge_index[0].astype(jnp.int32)
    dst = edge_index[1].astype(jnp.int32)
    ew = edge_weight.astype(_F32)

    npad = _EPAD - _E
    srcp = jnp.concatenate([src, jnp.zeros((npad,), jnp.int32)])
    dstp = jnp.concatenate([dst, jnp.zeros((npad,), jnp.int32)])
    ewp = jnp.concatenate([ew, jnp.zeros((npad,), _F32)])
    srcf = srcp.reshape(_NW, _EPW)
    dstf = dstp.reshape(_NW, _EPW)
    ewf = ewp.reshape(_NW, _EPW)
    src2 = srcp.reshape(_NW, _NCH, 128)
    dst2 = dstp.reshape(_NW, _NCH, 128)

    degp = _sc_deg(srcf, ewf)
    dis = _dis(degp).reshape(_N)
    wnf = _sc_wn(srcf, dstf, ewf, dis)

    def S(u):
        return jnp.stack([_sc_spmm(u[t], src2, dst2, wnf)
                          for t in range(_T1)])

    x2 = jnp.pad(x[0, :, :, 0].T, ((0, 0), (0, 16 - _TIN)))
    pad8 = lambda w: jnp.pad(w[:, 0, 0, :].T, ((0, 5), (0, 0)))
    h1 = _tconv1(x2, pad8(W1a), pad8(W1b), pad8(W1c),
                 b1a.reshape(1, _H), b1b.reshape(1, _H), b1c.reshape(1, _H))

    tx1 = _negsum(S(h1))
    s2 = S(tx1)
    tg = _cheb_combine(h1, tx1, s2, Wch, bch.reshape(1, _H))

    taps = lambda w: jnp.transpose(w[:, :, 0, :], (2, 1, 0))
    t2 = _tconv2(tg, taps(W2a), taps(W2b), taps(W2c),
                 b2a.reshape(1, _H), b2b.reshape(1, _H), b2c.reshape(1, _H))

    return _head(t2, gamma.reshape(_N, 1), beta.reshape(_N, 1),
                 Wlin, blin.reshape(1, 12))
